# baseline (speedup 1.0000x reference)
import functools

import jax
import jax.numpy as jnp
import numpy as np
from jax.experimental import pallas as pl
from jax.experimental.pallas import tpu as pltpu

EPS = np.float32(1e-6)
NEG = np.float32(-1e9)


def _sumsq_kernel(h_ref, o_ref):
    h = h_ref[...]
    o_ref[...] = jnp.sum(h * h, axis=-1, keepdims=True)


def _sumsq(h, block=256):
    S, D = h.shape
    return pl.pallas_call(
        _sumsq_kernel,
        grid=(S // block,),
        in_specs=[pl.BlockSpec((block, D), lambda i: (i, 0))],
        out_specs=pl.BlockSpec((block, 1), lambda i: (i, 0)),
        out_shape=jax.ShapeDtypeStruct((S, 1), jnp.float32),
    )(h)


def _rms_scale(ss, D):
    return jax.lax.rsqrt(ss * np.float32(1.0 / D) + EPS)


def _scale_kernel(h_ref, s_ref, g_ref, x_ref):
    x_ref[...] = h_ref[...] * s_ref[...] * g_ref[...]


def _apply_scale(h, scale, gamma, block=256):
    S, D = h.shape
    return pl.pallas_call(
        _scale_kernel,
        grid=(S // block,),
        in_specs=[
            pl.BlockSpec((block, D), lambda i: (i, 0)),
            pl.BlockSpec((block, 1), lambda i: (i, 0)),
            pl.BlockSpec((1, D), lambda i: (0, 0)),
        ],
        out_specs=pl.BlockSpec((block, D), lambda i: (i, 0)),
        out_shape=jax.ShapeDtypeStruct((S, D), jnp.float32),
    )(h, scale, gamma.reshape(1, D))


def _qkv_kernel(x_ref, wq_ref, wk_ref, wv_ref, q_ref, k_ref, v_ref):
    x = x_ref[...]
    for w_ref, o_ref in ((wq_ref, q_ref), (wk_ref, k_ref), (wv_ref, v_ref)):
        acc = jax.lax.dot(x, w_ref[...], preferred_element_type=jnp.float32)
        o_ref[...] = acc.astype(jnp.bfloat16)


def _qkv(x, Wq, Wk, Wv, bs=512, bd=512):
    S, D = x.shape
    grid = (D // bd, S // bs)
    return pl.pallas_call(
        _qkv_kernel,
        grid=grid,
        in_specs=[
            pl.BlockSpec((bs, D), lambda j, i: (i, 0)),
            pl.BlockSpec((D, bd), lambda j, i: (0, j)),
            pl.BlockSpec((D, bd), lambda j, i: (0, j)),
            pl.BlockSpec((D, bd), lambda j, i: (0, j)),
        ],
        out_specs=[
            pl.BlockSpec((bs, bd), lambda j, i: (i, j)),
            pl.BlockSpec((bs, bd), lambda j, i: (i, j)),
            pl.BlockSpec((bs, bd), lambda j, i: (i, j)),
        ],
        out_shape=[jax.ShapeDtypeStruct((S, D), jnp.bfloat16)] * 3,
    )(x, Wq, Wk, Wv)


def _attn_kernel(q_ref, k_ref, v_ref, o_ref, s_ref, *, bq, ck, hd):
    i = pl.program_id(1)
    q = q_ref[...]
    scale = np.float32(1.0 / np.sqrt(np.float64(hd)))
    rows = jax.lax.broadcasted_iota(jnp.int32, (bq, ck), 0) + i * bq

    def score_chunk(j, m):
        kc = k_ref[pl.ds(j * ck, ck), :]
        s = jax.lax.dot_general(q, kc, (((1,), (1,)), ((), ())),
                                preferred_element_type=jnp.float32) * scale
        cols = jax.lax.broadcasted_iota(jnp.int32, (bq, ck), 1) + j * ck
        s = jnp.where(cols <= rows, s, NEG)
        s_ref[:, pl.ds(j * ck, ck)] = s
        return jnp.maximum(m, jnp.max(s, axis=-1, keepdims=True))

    nj = i + 1
    m = jax.lax.fori_loop(0, nj, score_chunk,
                          jnp.full((bq, 1), -jnp.inf, jnp.float32))

    def exp_chunk(j, acc):
        e = jnp.exp(s_ref[:, pl.ds(j * ck, ck)] - m)
        s_ref[:, pl.ds(j * ck, ck)] = e
        return acc + jnp.sum(e, axis=-1, keepdims=True)

    denom = jax.lax.fori_loop(0, nj, exp_chunk,
                              jnp.zeros((bq, 1), jnp.float32))

    def pv_chunk(j, o):
        p = (s_ref[:, pl.ds(j * ck, ck)] / denom).astype(jnp.bfloat16)
        vc = v_ref[pl.ds(j * ck, ck), :]
        return o + jax.lax.dot(p, vc, preferred_element_type=jnp.float32)

    o = jax.lax.fori_loop(0, nj, pv_chunk,
                          jnp.zeros((bq, hd), jnp.float32))
    o_ref[...] = o.astype(jnp.bfloat16)


def _attention(q, k, v, n_heads, hd, bq=256, ck=256):
    S, D = q.shape
    kern = functools.partial(_attn_kernel, bq=bq, ck=ck, hd=hd)
    return pl.pallas_call(
        kern,
        grid=(n_heads, S // bq),
        in_specs=[
            pl.BlockSpec((bq, hd), lambda h, i: (i, h)),
            pl.BlockSpec((S, hd), lambda h, i: (0, h)),
            pl.BlockSpec((S, hd), lambda h, i: (0, h)),
        ],
        out_specs=pl.BlockSpec((bq, hd), lambda h, i: (i, h)),
        out_shape=jax.ShapeDtypeStruct((S, D), jnp.bfloat16),
        scratch_shapes=[pltpu.VMEM((bq, S), jnp.float32)],
    )(q, k, v)


def _proj_res_kernel(attn_ref, wo_ref, res_ref, hid_ref):
    acc = jax.lax.dot(attn_ref[...], wo_ref[...],
                      preferred_element_type=jnp.float32)
    hid_ref[...] = acc + res_ref[...]


def _proj_res(attn, Wo, resid, bs=512, bd=512):
    S, D = resid.shape
    return pl.pallas_call(
        _proj_res_kernel,
        grid=(D // bd, S // bs),
        in_specs=[
            pl.BlockSpec((bs, D), lambda j, i: (i, 0)),
            pl.BlockSpec((D, bd), lambda j, i: (0, j)),
            pl.BlockSpec((bs, bd), lambda j, i: (i, j)),
        ],
        out_specs=pl.BlockSpec((bs, bd), lambda j, i: (i, j)),
        out_shape=jax.ShapeDtypeStruct((S, D), jnp.float32),
    )(attn, Wo, resid)


def _router_kernel(hid_ref, s_ref, g_ref, wg_ref, log_ref):
    x2 = hid_ref[...] * s_ref[...] * g_ref[...]
    log_ref[...] = jax.lax.dot(x2, wg_ref[...],
                               preferred_element_type=jnp.float32)


def _router_logits(hidden, scale, gamma, W_gate, block=256):
    S, D = hidden.shape
    E = W_gate.shape[1]
    return pl.pallas_call(
        _router_kernel,
        grid=(S // block,),
        in_specs=[
            pl.BlockSpec((block, D), lambda i: (i, 0)),
            pl.BlockSpec((block, 1), lambda i: (i, 0)),
            pl.BlockSpec((1, D), lambda i: (0, 0)),
            pl.BlockSpec((D, E), lambda i: (0, 0)),
        ],
        out_specs=pl.BlockSpec((block, E), lambda i: (i, 0)),
        out_shape=jax.ShapeDtypeStruct((S, E), jnp.float32),
    )(hidden, scale, gamma.reshape(1, D), W_gate)


def _topk_kernel(log_ref, w_ref, id_ref, *, top_k):
    logits = log_ref[...]
    m = jnp.max(logits, axis=-1, keepdims=True)
    e = jnp.exp(logits - m)
    p = e / jnp.sum(e, axis=-1, keepdims=True)
    bs, E = p.shape
    lanes = jax.lax.broadcasted_iota(jnp.int32, (bs, E), 1)
    ws, ids = [], []
    for _ in range(top_k):
        mx = jnp.max(p, axis=-1, keepdims=True)
        hit = p == mx
        idx = jnp.min(jnp.where(hit, lanes, E), axis=-1, keepdims=True)
        ws.append(mx)
        ids.append(idx)
        p = jnp.where(lanes == idx, jnp.float32(-1.0), p)
    w = jnp.concatenate(ws, axis=-1)
    w_ref[...] = w / jnp.sum(w, axis=-1, keepdims=True)
    id_ref[...] = jnp.concatenate(ids, axis=-1)


def _topk(logits, top_k, block=256):
    S, E = logits.shape
    kern = functools.partial(_topk_kernel, top_k=top_k)
    return pl.pallas_call(
        kern,
        grid=(S // block,),
        in_specs=[pl.BlockSpec((block, E), lambda i: (i, 0))],
        out_specs=[
            pl.BlockSpec((block, top_k), lambda i: (i, 0)),
            pl.BlockSpec((block, top_k), lambda i: (i, 0)),
        ],
        out_shape=[
            jax.ShapeDtypeStruct((S, top_k), jnp.float32),
            jax.ShapeDtypeStruct((S, top_k), jnp.int32),
        ],
    )(logits)


def kernel(hidden_states, pre_ln_gamma, post_ln_gamma, Wq, Wk, Wv, Wo, W_gate):
    S, D = hidden_states.shape
    n_heads = 16
    hd = D // n_heads
    top_k = 8

    scale1 = _rms_scale(_sumsq(hidden_states), D)
    x = _apply_scale(hidden_states, scale1, pre_ln_gamma)
    q, k, v = _qkv(x, Wq, Wk, Wv)
    attn = _attention(q, k, v, n_heads, hd)
    hidden = _proj_res(attn, Wo, hidden_states)
    scale2 = _rms_scale(_sumsq(hidden), D)
    logits = _router_logits(hidden, scale2, post_ln_gamma, W_gate)
    topk_w, topk_i = _topk(logits, top_k)
    return hidden, topk_w, topk_i
